# baseline (speedup 1.0000x reference)
import functools

import jax
import jax.numpy as jnp
from jax import lax
from jax.experimental import pallas as pl
from jax.experimental.pallas import tpu as pltpu
from jax.experimental.pallas import tpu_sc as plsc

_TILE = 512
_E = 64
_GPB = 2


def _half_top2(gh, off):
    n = gh.shape[-1]
    ids = jax.lax.broadcasted_iota(jnp.int32, gh.shape, 1) + off
    m1 = jnp.max(gh, axis=-1, keepdims=True)
    i1 = jnp.min(jnp.where(gh == m1, ids, _E), axis=-1, keepdims=True)
    g2 = jnp.where(ids == i1, -1.0, gh)
    m2 = jnp.max(g2, axis=-1, keepdims=True)
    i2 = jnp.min(jnp.where(g2 == m2, ids, _E), axis=-1, keepdims=True)
    return m1, i1, m2, i2


def _router_tc_body(xa_ref, xb_ref, wt_ref, b_ref, gate_ref, cv_ref, ci_ref):
    h = wt_ref.shape[0] // 2
    logits = (jnp.dot(xa_ref[...], wt_ref[:h],
                      preferred_element_type=jnp.float32)
              + jnp.dot(xb_ref[...], wt_ref[h:],
                        preferred_element_type=jnp.float32)
              + b_ref[...])
    m = jnp.max(logits, axis=-1, keepdims=True)
    e = jnp.exp(logits - m)
    s = jnp.sum(e, axis=-1, keepdims=True)
    g = e / s
    gate_ref[...] = g
    he = _E // 2
    a1, ia1, a2, ia2 = _half_top2(g[:, :he], 0)
    b1, ib1, b2, ib2 = _half_top2(g[:, he:], he)
    cv = jnp.concatenate([a1, a2, b1, b2], axis=-1)
    ci = jnp.concatenate([ia1, ia2, ib1, ib2], axis=-1)
    cv_ref[...] = cv.T.reshape(1, 4, _TILE)
    ci_ref[...] = ci.T.reshape(1, 4, _TILE)


def _make_sc_merge(n_tokens):
    info = plsc.get_sparse_core_info()
    nc, ns, L = info.num_cores, info.num_subcores, info.num_lanes
    nw = nc * ns
    tpw = n_tokens // nw
    mesh = plsc.VectorSubcoreMesh(core_axis_name="c", subcore_axis_name="s",
                                  num_cores=nc)

    def body(cv_ref, ci_ref, v1_ref, v2_ref, i1_ref, i2_ref,
             cv_v, ci_v, v1_v, v2_v, i1_v, i2_v):
        wid = lax.axis_index("s") * nc + lax.axis_index("c")
        base = wid * tpw
        pltpu.sync_copy(cv_ref.at[wid], cv_v)
        pltpu.sync_copy(ci_ref.at[wid], ci_v)

        def group_body(g, carry):
            for k in range(_GPB):
                o = g * (_GPB * L) + k * L
                a1 = cv_v[pl.ds(0 * tpw + o, L)]
                a2 = cv_v[pl.ds(1 * tpw + o, L)]
                b1 = cv_v[pl.ds(2 * tpw + o, L)]
                b2 = cv_v[pl.ds(3 * tpw + o, L)]
                ia1 = ci_v[pl.ds(0 * tpw + o, L)]
                ia2 = ci_v[pl.ds(1 * tpw + o, L)]
                ib1 = ci_v[pl.ds(2 * tpw + o, L)]
                ib2 = ci_v[pl.ds(3 * tpw + o, L)]
                c1 = b1 > a1
                c2 = b2 > a1
                c3 = b1 > a2
                m1 = jnp.where(c1, b1, a1)
                i1 = jnp.where(c1, ib1, ia1)
                m2 = jnp.where(c1, jnp.where(c2, b2, a1),
                               jnp.where(c3, b1, a2))
                i2 = jnp.where(c1, jnp.where(c2, ib2, ia1),
                               jnp.where(c3, ib1, ia2))
                v1_v[pl.ds(o, L)] = m1
                v2_v[pl.ds(o, L)] = m2
                i1_v[pl.ds(o, L)] = i1
                i2_v[pl.ds(o, L)] = i2
            return carry

        lax.fori_loop(0, tpw // (_GPB * L), group_body, 0)
        pltpu.sync_copy(v1_v, v1_ref.at[pl.ds(base, tpw)])
        pltpu.sync_copy(v2_v, v2_ref.at[pl.ds(base, tpw)])
        pltpu.sync_copy(i1_v, i1_ref.at[pl.ds(base, tpw)])
        pltpu.sync_copy(i2_v, i2_ref.at[pl.ds(base, tpw)])

    vec = jax.ShapeDtypeStruct((n_tokens,), jnp.float32)
    ivec = jax.ShapeDtypeStruct((n_tokens,), jnp.int32)
    return nw, tpw, functools.partial(
        pl.kernel, mesh=mesh,
        out_type=[vec, vec, ivec, ivec],
        scratch_types=[
            pltpu.VMEM((4 * tpw,), jnp.float32),
            pltpu.VMEM((4 * tpw,), jnp.int32),
            pltpu.VMEM((tpw,), jnp.float32),
            pltpu.VMEM((tpw,), jnp.float32),
            pltpu.VMEM((tpw,), jnp.int32),
            pltpu.VMEM((tpw,), jnp.int32),
        ],
    )(body)


@jax.jit
def kernel(x, W, b):
    B, S, D = x.shape
    E = W.shape[0]
    N = B * S
    xf = x.reshape(N, D)
    wt = W.T
    b2 = b.reshape(1, E)
    grid = (N // _TILE,)
    gate, cand_v, cand_i = pl.pallas_call(
        _router_tc_body,
        grid=grid,
        in_specs=[
            pl.BlockSpec((_TILE, D // 2), lambda i: (i, 0)),
            pl.BlockSpec((_TILE, D // 2), lambda i: (i, 1)),
            pl.BlockSpec((D, E), lambda i: (0, 0)),
            pl.BlockSpec((1, E), lambda i: (0, 0)),
        ],
        out_specs=[
            pl.BlockSpec((_TILE, E), lambda i: (i, 0)),
            pl.BlockSpec((1, 4, _TILE), lambda i: (i, 0, 0)),
            pl.BlockSpec((1, 4, _TILE), lambda i: (i, 0, 0)),
        ],
        out_shape=[
            jax.ShapeDtypeStruct((N, E), jnp.float32),
            jax.ShapeDtypeStruct((N // _TILE, 4, _TILE), jnp.float32),
            jax.ShapeDtypeStruct((N // _TILE, 4, _TILE), jnp.int32),
        ],
    )(xf, xf, wt, b2)

    nw, tpw, sc_merge = _make_sc_merge(N)
    v1, v2, i1, i2 = sc_merge(cand_v.reshape(nw, 4 * tpw),
                              cand_i.reshape(nw, 4 * tpw))

    top2_val = jnp.stack([v1, v2], axis=-1).reshape(B, S, 2)
    top2_idx = jnp.stack([i1, i2], axis=-1).reshape(B, S, 2).astype(jnp.int64)
    return (top2_val, top2_idx, gate.reshape(B, S, E))


# speedup vs baseline: 1.0373x; 1.0373x over previous
import functools

import jax
import jax.numpy as jnp
from jax import lax
from jax.experimental import pallas as pl
from jax.experimental.pallas import tpu as pltpu
from jax.experimental.pallas import tpu_sc as plsc

_TILE = 256
_E = 64
_GPB = 4
_CHUNKS = 2


def _tc_body_first(xa_ref, xb_ref, wt_ref, b_ref, gate_ref, gsc_ref):
    _tc_compute(xa_ref, xb_ref, wt_ref, b_ref, gate_ref, gsc_ref)


def _tc_body_next(xa_ref, xb_ref, wt_ref, b_ref, gin_ref, gate_ref, gsc_ref):
    _tc_compute(xa_ref, xb_ref, wt_ref, b_ref, gate_ref, gsc_ref)


def _tc_compute(xa_ref, xb_ref, wt_ref, b_ref, gate_ref, gsc_ref):
    h = wt_ref.shape[0] // 2
    logits = (jnp.dot(xa_ref[...], wt_ref[:h],
                      preferred_element_type=jnp.float32)
              + jnp.dot(xb_ref[...], wt_ref[h:],
                        preferred_element_type=jnp.float32)
              + b_ref[...])
    m = jnp.max(logits, axis=-1, keepdims=True)
    e = jnp.exp(logits - m)
    s = jnp.sum(e, axis=-1, keepdims=True)
    g = e / s
    gate_ref[...] = g
    gsc_ref[...] = g.T.reshape(1, _E, _TILE)


def _make_sc_top2(n_tokens):
    info = plsc.get_sparse_core_info()
    nc, ns, L = info.num_cores, info.num_subcores, info.num_lanes
    nw = nc * ns
    tpw = n_tokens // nw
    mesh = plsc.VectorSubcoreMesh(core_axis_name="c", subcore_axis_name="s",
                                  num_cores=nc)

    def body(gs_ref, v1_ref, v2_ref, i1_ref, i2_ref,
             gt_v, v1_v, v2_v, i1_v, i2_v):
        wid = lax.axis_index("s") * nc + lax.axis_index("c")
        base = wid * tpw
        pltpu.sync_copy(gs_ref.at[wid], gt_v)

        def group_body(g, carry):
            offs = [g * (_GPB * L) + k * L for k in range(_GPB)]
            m1 = [gt_v[pl.ds(o, L)] for o in offs]
            i1 = [jnp.zeros((L,), jnp.int32) for _ in offs]
            m2 = [jnp.full((L,), -1.0, jnp.float32) for _ in offs]
            i2 = [jnp.zeros((L,), jnp.int32) for _ in offs]
            for e in range(1, _E):
                ev = jnp.full((L,), e, jnp.int32)
                for k, o in enumerate(offs):
                    v = gt_v[pl.ds(e * tpw + o, L)]
                    gt1 = v > m1[k]
                    gt2 = v > m2[k]
                    nm2 = jnp.where(gt1, m1[k], jnp.where(gt2, v, m2[k]))
                    ni2 = jnp.where(gt1, i1[k], jnp.where(gt2, ev, i2[k]))
                    m1[k] = jnp.where(gt1, v, m1[k])
                    i1[k] = jnp.where(gt1, ev, i1[k])
                    m2[k] = nm2
                    i2[k] = ni2
            for k, o in enumerate(offs):
                v1_v[pl.ds(o, L)] = m1[k]
                v2_v[pl.ds(o, L)] = m2[k]
                i1_v[pl.ds(o, L)] = i1[k]
                i2_v[pl.ds(o, L)] = i2[k]
            return carry

        lax.fori_loop(0, tpw // (_GPB * L), group_body, 0)
        pltpu.sync_copy(v1_v, v1_ref.at[pl.ds(base, tpw)])
        pltpu.sync_copy(v2_v, v2_ref.at[pl.ds(base, tpw)])
        pltpu.sync_copy(i1_v, i1_ref.at[pl.ds(base, tpw)])
        pltpu.sync_copy(i2_v, i2_ref.at[pl.ds(base, tpw)])

    vec = jax.ShapeDtypeStruct((n_tokens,), jnp.float32)
    ivec = jax.ShapeDtypeStruct((n_tokens,), jnp.int32)
    return nw, tpw, functools.partial(
        pl.kernel, mesh=mesh,
        out_type=[vec, vec, ivec, ivec],
        scratch_types=[
            pltpu.VMEM((_E * tpw,), jnp.float32),
            pltpu.VMEM((tpw,), jnp.float32),
            pltpu.VMEM((tpw,), jnp.float32),
            pltpu.VMEM((tpw,), jnp.int32),
            pltpu.VMEM((tpw,), jnp.int32),
        ],
    )(body)


@jax.jit
def kernel(x, W, b):
    B, S, D = x.shape
    E = W.shape[0]
    N = B * S
    NC_TOK = N // _CHUNKS
    xf = x.reshape(N, D)
    wt = W.T
    b2 = b.reshape(1, E)
    blocks_per_chunk = NC_TOK // _TILE

    nw, tpw, sc_top2 = _make_sc_top2(NC_TOK)
    assert blocks_per_chunk == nw

    x_spec_a = lambda c: pl.BlockSpec(
        (_TILE, D // 2), lambda i, c=c: (c * blocks_per_chunk + i, 0))
    x_spec_b = lambda c: pl.BlockSpec(
        (_TILE, D // 2), lambda i, c=c: (c * blocks_per_chunk + i, 1))
    gate_spec = lambda c: pl.BlockSpec(
        (_TILE, E), lambda i, c=c: (c * blocks_per_chunk + i, 0))
    gsc_spec = pl.BlockSpec((1, E, _TILE), lambda i: (i, 0, 0))
    gsc_shape = jax.ShapeDtypeStruct((blocks_per_chunk, E, _TILE),
                                     jnp.float32)

    gate, gsc0 = pl.pallas_call(
        _tc_body_first,
        grid=(blocks_per_chunk,),
        in_specs=[x_spec_a(0), x_spec_b(0),
                  pl.BlockSpec((D, E), lambda i: (0, 0)),
                  pl.BlockSpec((1, E), lambda i: (0, 0))],
        out_specs=[gate_spec(0), gsc_spec],
        out_shape=[jax.ShapeDtypeStruct((N, E), jnp.float32), gsc_shape],
    )(xf, xf, wt, b2)

    outs = [sc_top2(gsc0.reshape(nw, E * tpw))]
    for c in range(1, _CHUNKS):
        gate, gsc = pl.pallas_call(
            _tc_body_next,
            grid=(blocks_per_chunk,),
            in_specs=[x_spec_a(c), x_spec_b(c),
                      pl.BlockSpec((D, E), lambda i: (0, 0)),
                      pl.BlockSpec((1, E), lambda i: (0, 0)),
                      pl.BlockSpec(memory_space=pl.ANY)],
            out_specs=[gate_spec(c), gsc_spec],
            out_shape=[jax.ShapeDtypeStruct((N, E), jnp.float32), gsc_shape],
            input_output_aliases={4: 0},
        )(xf, xf, wt, b2, gate)
        outs.append(sc_top2(gsc.reshape(nw, E * tpw)))

    v1, v2, i1, i2 = (jnp.concatenate([o[j] for o in outs]) for j in range(4))
    top2_val = jnp.stack([v1, v2], axis=-1).reshape(B, S, 2)
    top2_idx = jnp.stack([i1, i2], axis=-1).reshape(B, S, 2).astype(jnp.int64)
    return (top2_val, top2_idx, gate.reshape(B, S, E))


# speedup vs baseline: 1.2999x; 1.2531x over previous
import functools

import jax
import jax.numpy as jnp
from jax import lax
from jax.experimental import pallas as pl
from jax.experimental.pallas import tpu as pltpu
from jax.experimental.pallas import tpu_sc as plsc

_TILE = 512
_E = 64
_GPB = 2


def _router_tc_body(xa_ref, xb_ref, wt_ref, b_ref, gate_ref, gsc_ref):
    h = wt_ref.shape[0] // 2
    logits = (jnp.dot(xa_ref[...], wt_ref[:h],
                      preferred_element_type=jnp.float32)
              + jnp.dot(xb_ref[...], wt_ref[h:],
                        preferred_element_type=jnp.float32)
              + b_ref[...])
    m = jnp.max(logits, axis=-1, keepdims=True)
    e = jnp.exp(logits - m)
    s = jnp.sum(e, axis=-1, keepdims=True)
    g = e / s
    gate_ref[...] = g
    gsc_ref[...] = g.T.reshape(1, _E, _TILE)


def _make_sc_top2(n_tokens):
    info = plsc.get_sparse_core_info()
    nc, ns, L = info.num_cores, info.num_subcores, info.num_lanes
    nw = nc * ns
    tpw = n_tokens // nw
    mesh = plsc.VectorSubcoreMesh(core_axis_name="c", subcore_axis_name="s",
                                  num_cores=nc)

    def body(gs_ref, v1_ref, v2_ref, i1_ref, i2_ref,
             gt_v, v1_v, v2_v, i1_v, i2_v):
        wid = lax.axis_index("s") * nc + lax.axis_index("c")
        base = wid * tpw
        pltpu.sync_copy(gs_ref.at[wid], gt_v)

        def group_body(g, carry):
            offs = [g * (_GPB * L) + k * L for k in range(_GPB)]
            m1 = [gt_v[pl.ds(o, L)] for o in offs]
            i1 = [jnp.zeros((L,), jnp.int32) for _ in offs]
            m2 = [jnp.full((L,), -1.0, jnp.float32) for _ in offs]
            i2 = [jnp.zeros((L,), jnp.int32) for _ in offs]
            for e in range(1, _E):
                ev = jnp.full((L,), e, jnp.int32)
                for k, o in enumerate(offs):
                    v = gt_v[pl.ds(e * tpw + o, L)]
                    gt1 = v > m1[k]
                    gt2 = v > m2[k]
                    nm2 = jnp.where(gt1, m1[k], jnp.where(gt2, v, m2[k]))
                    ni2 = jnp.where(gt1, i1[k], jnp.where(gt2, ev, i2[k]))
                    m1[k] = jnp.where(gt1, v, m1[k])
                    i1[k] = jnp.where(gt1, ev, i1[k])
                    m2[k] = nm2
                    i2[k] = ni2
            for k, o in enumerate(offs):
                v1_v[pl.ds(o, L)] = m1[k]
                v2_v[pl.ds(o, L)] = m2[k]
                i1_v[pl.ds(o, L)] = i1[k]
                i2_v[pl.ds(o, L)] = i2[k]
            return carry

        lax.fori_loop(0, tpw // (_GPB * L), group_body, 0)
        pltpu.sync_copy(v1_v, v1_ref.at[pl.ds(base, tpw)])
        pltpu.sync_copy(v2_v, v2_ref.at[pl.ds(base, tpw)])
        pltpu.sync_copy(i1_v, i1_ref.at[pl.ds(base, tpw)])
        pltpu.sync_copy(i2_v, i2_ref.at[pl.ds(base, tpw)])

    vec = jax.ShapeDtypeStruct((n_tokens,), jnp.float32)
    ivec = jax.ShapeDtypeStruct((n_tokens,), jnp.int32)
    return nw, tpw, functools.partial(
        pl.kernel, mesh=mesh,
        out_type=[vec, vec, ivec, ivec],
        scratch_types=[
            pltpu.VMEM((_E * tpw,), jnp.float32),
            pltpu.VMEM((tpw,), jnp.float32),
            pltpu.VMEM((tpw,), jnp.float32),
            pltpu.VMEM((tpw,), jnp.int32),
            pltpu.VMEM((tpw,), jnp.int32),
        ],
    )(body)


@jax.jit
def kernel(x, W, b):
    B, S, D = x.shape
    E = W.shape[0]
    N = B * S
    xf = x.reshape(N, D)
    wt = W.T
    b2 = b.reshape(1, E)
    grid = (N // _TILE,)
    gate, gate_sc = pl.pallas_call(
        _router_tc_body,
        grid=grid,
        in_specs=[
            pl.BlockSpec((_TILE, D // 2), lambda i: (i, 0)),
            pl.BlockSpec((_TILE, D // 2), lambda i: (i, 1)),
            pl.BlockSpec((D, E), lambda i: (0, 0)),
            pl.BlockSpec((1, E), lambda i: (0, 0)),
        ],
        out_specs=[
            pl.BlockSpec((_TILE, E), lambda i: (i, 0)),
            pl.BlockSpec((1, E, _TILE), lambda i: (i, 0, 0)),
        ],
        out_shape=[
            jax.ShapeDtypeStruct((N, E), jnp.float32),
            jax.ShapeDtypeStruct((N // _TILE, E, _TILE), jnp.float32),
        ],
    )(xf, xf, wt, b2)

    nw, tpw, sc_top2 = _make_sc_top2(N)
    gs = gate_sc.reshape(nw, E * tpw)
    v1, v2, i1, i2 = sc_top2(gs)

    top2_val = jnp.stack([v1, v2], axis=-1).reshape(B, S, 2)
    top2_idx = jnp.stack([i1, i2], axis=-1).reshape(B, S, 2).astype(jnp.int64)
    return (top2_val, top2_idx, gate.reshape(B, S, E))
